# baseline (speedup 1.0000x reference)
import functools

import jax
import jax.numpy as jnp
from jax import lax
from jax.experimental import pallas as pl
from jax.experimental.pallas import tpu as pltpu
from jax.experimental.pallas import tpu_sc as plsc

V = 1000000
D = 64
B = 4096
L = 200
NC = 2
NS = 16
NW = NC * NS

FULL_WC = 7812
PAIRS = V // 2

_MESH = plsc.VectorSubcoreMesh(core_axis_name="c", subcore_axis_name="s")
_PARAMS = pltpu.CompilerParams(use_tc_tiling_on_sc=True,
                               needs_layout_passes=False)


def _transpose_block(inbuf, outbuf, npairs, iota16, wbase=0):
    for p in range(npairs):
        for k in range(8):
            a = k // 4
            f16 = iota16 + (k % 4) * 16
            w16 = jnp.full((16,), wbase + 2 * p + a, jnp.int32)
            outbuf[p, pl.ds(k * 16, 16)] = plsc.load_gather(inbuf, [f16, w16])


@functools.partial(
    pl.kernel,
    mesh=_MESH,
    out_type=jax.ShapeDtypeStruct((PAIRS, 128), jnp.float32),
    scratch_types=[
        pltpu.VMEM((64, 128), jnp.float32),
        pltpu.VMEM((64, 128), jnp.float32),
        pltpu.VMEM((64, 128), jnp.float32),
        pltpu.VMEM((64, 128), jnp.float32),
        pltpu.SemaphoreType.DMA,
        pltpu.SemaphoreType.DMA,
        pltpu.SemaphoreType.DMA,
        pltpu.SemaphoreType.DMA,
    ],
    compiler_params=_PARAMS,
)
def _k1(tT_hbm, tail_hbm, scratch_hbm, in0, in1, out0, out1,
        gs0, gs1, os0, os1):
    wid = lax.axis_index("s") * NC + lax.axis_index("c")
    iota16 = lax.iota(jnp.int32, 16)

    def wc_of(t):
        return wid + NW * t

    def start_in(t, buf, sem):
        @pl.when(wc_of(t) < FULL_WC)
        def _():
            off = pl.multiple_of(wc_of(t) * 128, 128)
            pltpu.async_copy(tT_hbm.at[:, pl.ds(off, 128)], buf, sem)

    def wait_in(t, buf, sem):
        @pl.when(wc_of(t) < FULL_WC)
        def _():
            pltpu.make_async_copy(
                tT_hbm.at[:, pl.ds(0, 128)], buf, sem).wait()

    def start_out(t, buf, sem):
        @pl.when(wc_of(t) < FULL_WC)
        def _():
            off = pl.multiple_of(wc_of(t) * 64, 64)
            pltpu.async_copy(buf, scratch_hbm.at[pl.ds(off, 64)], sem)

    def wait_out(t, buf, sem):
        @pl.when(wc_of(t) < FULL_WC)
        def _():
            pltpu.make_async_copy(
                buf, scratch_hbm.at[pl.ds(0, 64)], sem).wait()

    def compute(t, inbuf, outbuf):
        @pl.when(wc_of(t) < FULL_WC)
        def _():
            _transpose_block(inbuf, outbuf, 64, iota16)

    start_in(0, in0, gs0)

    def body(tt, carry):
        t0 = 2 * tt
        t1 = 2 * tt + 1
        start_in(t1, in1, gs1)
        wait_in(t0, in0, gs0)

        @pl.when(tt > 0)
        def _():
            wait_out(2 * tt - 2, out0, os0)
        compute(t0, in0, out0)
        start_out(t0, out0, os0)

        start_in(t0 + 2, in0, gs0)
        wait_in(t1, in1, gs1)

        @pl.when(tt > 0)
        def _():
            wait_out(2 * tt - 1, out1, os1)
        compute(t1, in1, out1)
        start_out(t1, out1, os1)
        return carry

    lax.fori_loop(0, 123, body, 0)
    wait_out(244, out0, os0)
    wait_out(245, out1, os1)

    @pl.when(wid == 0)
    def _():
        pltpu.sync_copy(tail_hbm, in0)
        _transpose_block(in0, out0, 32, iota16, wbase=64)
        pltpu.sync_copy(out0.at[pl.ds(0, 32)],
                        scratch_hbm.at[pl.ds(FULL_WC * 64, 32)])


@functools.partial(
    pl.kernel,
    mesh=_MESH,
    out_type=jax.ShapeDtypeStruct((L, 8, 32, 8, 128), jnp.float32),
    scratch_types=[
        pltpu.VMEM((128,), jnp.int32),
        pltpu.VMEM((128,), jnp.int32),
        pltpu.VMEM((128,), jnp.int32),
        pltpu.VMEM((128,), jnp.int32),
        pltpu.VMEM((128, 128), jnp.float32),
        pltpu.VMEM((128, 128), jnp.float32),
        pltpu.VMEM((8, 8, 128), jnp.float32),
        pltpu.VMEM((8, 8, 128), jnp.float32),
        pltpu.SemaphoreType.DMA,
        pltpu.SemaphoreType.DMA,
        pltpu.SemaphoreType.DMA,
        pltpu.SemaphoreType.DMA,
    ],
    compiler_params=_PARAMS,
)
def _k2(xT_hbm, scratch_hbm, out_hbm,
        idx0, idx1, pidx0, pidx1, pairs0, pairs1, tb0, tb1,
        gs0, gs1, os0, os1):
    wid = lax.axis_index("s") * NC + lax.axis_index("c")
    iota16 = lax.iota(jnp.int32, 16)

    def prep(l, idx_v, pidx_v, pairs_v, gsem):
        boff = pl.multiple_of(wid * 128, 128)
        pltpu.sync_copy(xT_hbm.at[l, pl.ds(boff, 128)], idx_v)
        for g in range(8):
            pidx_v[pl.ds(g * 16, 16)] = (
                lax.shift_right_logical(idx_v[pl.ds(g * 16, 16)], 1))
        pltpu.async_copy(scratch_hbm.at[pidx_v], pairs_v, gsem)

    def wait_gather(pairs_v, gsem):
        pltpu.make_async_copy(
            scratch_hbm.at[pl.ds(0, 128)], pairs_v, gsem).wait()

    def compact(idx_v, pairs_v, tbuf):
        for g in range(8):
            r16 = iota16 + g * 16
            col0 = (idx_v[pl.ds(g * 16, 16)] & 1) * 64
            for m in range(64):
                tbuf[m // 8, m % 8, pl.ds(g * 16, 16)] = plsc.load_gather(
                    pairs_v, [r16, col0 + m])

    def start_out(l, tbuf, sem):
        pltpu.async_copy(tbuf, out_hbm.at[l, :, wid, :, :], sem)

    def wait_out(tbuf, sem):
        pltpu.make_async_copy(tbuf, out_hbm.at[0, :, wid, :, :], sem).wait()

    prep(0, idx0, pidx0, pairs0, gs0)

    def body(ll, carry):
        l0 = 2 * ll
        l1 = 2 * ll + 1
        prep(l1, idx1, pidx1, pairs1, gs1)
        wait_gather(pairs0, gs0)

        @pl.when(ll > 0)
        def _():
            wait_out(tb0, os0)
        compact(idx0, pairs0, tb0)
        start_out(l0, tb0, os0)

        @pl.when(ll < (L // 2) - 1)
        def _():
            prep(l0 + 2, idx0, pidx0, pairs0, gs0)
        wait_gather(pairs1, gs1)

        @pl.when(ll > 0)
        def _():
            wait_out(tb1, os1)
        compact(idx1, pairs1, tb1)
        start_out(l1, tb1, os1)
        return carry

    lax.fori_loop(0, L // 2, body, 0)
    wait_out(tb0, os0)
    wait_out(tb1, os1)


def kernel(x, table):
    tT = table.T
    tail = tT[:, V - 128:]
    scratch = _k1(tT, tail)
    xT = x.astype(jnp.int32).T
    out5 = _k2(xT, scratch)
    return out5.transpose(2, 4, 0, 1, 3).reshape(B, L, D)


# speedup vs baseline: 1.4197x; 1.4197x over previous
import functools

import jax
import jax.numpy as jnp
from jax import lax
from jax.experimental import pallas as pl
from jax.experimental.pallas import tpu as pltpu
from jax.experimental.pallas import tpu_sc as plsc

V = 1000000
D = 64
B = 4096
L = 200
NC = 2
NS = 16
NW = NC * NS

FULL_WC = 7812
PAIRS = V // 2

_MESH = plsc.VectorSubcoreMesh(core_axis_name="c", subcore_axis_name="s")
_PARAMS = pltpu.CompilerParams(use_tc_tiling_on_sc=True,
                               needs_layout_passes=False)


def _transpose_block(inbuf, outbuf, npairs, iota16, wbase=0):
    for p in range(npairs):
        vals = []
        for k in range(8):
            a = k // 4
            f16 = iota16 + (k % 4) * 16
            w16 = jnp.full((16,), wbase + 2 * p + a, jnp.int32)
            vals.append(plsc.load_gather(inbuf, [f16, w16]))
        for k in range(8):
            outbuf[p, pl.ds(k * 16, 16)] = vals[k]


@functools.partial(
    pl.kernel,
    mesh=_MESH,
    out_type=jax.ShapeDtypeStruct((PAIRS, 128), jnp.float32),
    scratch_types=[
        pltpu.VMEM((64, 128), jnp.float32),
        pltpu.VMEM((64, 128), jnp.float32),
        pltpu.VMEM((64, 128), jnp.float32),
        pltpu.VMEM((64, 128), jnp.float32),
        pltpu.SemaphoreType.DMA,
        pltpu.SemaphoreType.DMA,
        pltpu.SemaphoreType.DMA,
        pltpu.SemaphoreType.DMA,
    ],
    compiler_params=_PARAMS,
)
def _k1(tT_hbm, tail_hbm, scratch_hbm, in0, in1, out0, out1,
        gs0, gs1, os0, os1):
    wid = lax.axis_index("s") * NC + lax.axis_index("c")
    iota16 = lax.iota(jnp.int32, 16)

    def wc_of(t):
        return wid + NW * t

    def start_in(t, buf, sem):
        @pl.when(wc_of(t) < FULL_WC)
        def _():
            off = pl.multiple_of(wc_of(t) * 128, 128)
            pltpu.async_copy(tT_hbm.at[:, pl.ds(off, 128)], buf, sem)

    def wait_in(t, buf, sem):
        @pl.when(wc_of(t) < FULL_WC)
        def _():
            pltpu.make_async_copy(
                tT_hbm.at[:, pl.ds(0, 128)], buf, sem).wait()

    def start_out(t, buf, sem):
        @pl.when(wc_of(t) < FULL_WC)
        def _():
            off = pl.multiple_of(wc_of(t) * 64, 64)
            pltpu.async_copy(buf, scratch_hbm.at[pl.ds(off, 64)], sem)

    def wait_out(t, buf, sem):
        @pl.when(wc_of(t) < FULL_WC)
        def _():
            pltpu.make_async_copy(
                buf, scratch_hbm.at[pl.ds(0, 64)], sem).wait()

    def compute(t, inbuf, outbuf):
        @pl.when(wc_of(t) < FULL_WC)
        def _():
            _transpose_block(inbuf, outbuf, 64, iota16)

    start_in(0, in0, gs0)

    def body(tt, carry):
        t0 = 2 * tt
        t1 = 2 * tt + 1
        start_in(t1, in1, gs1)
        wait_in(t0, in0, gs0)

        @pl.when(tt > 0)
        def _():
            wait_out(2 * tt - 2, out0, os0)
        compute(t0, in0, out0)
        start_out(t0, out0, os0)

        start_in(t0 + 2, in0, gs0)
        wait_in(t1, in1, gs1)

        @pl.when(tt > 0)
        def _():
            wait_out(2 * tt - 1, out1, os1)
        compute(t1, in1, out1)
        start_out(t1, out1, os1)
        return carry

    lax.fori_loop(0, 123, body, 0)
    wait_out(244, out0, os0)
    wait_out(245, out1, os1)

    @pl.when(wid == 0)
    def _():
        pltpu.sync_copy(tail_hbm, in0)
        _transpose_block(in0, out0, 32, iota16, wbase=64)
        pltpu.sync_copy(out0.at[pl.ds(0, 32)],
                        scratch_hbm.at[pl.ds(FULL_WC * 64, 32)])


@functools.partial(
    pl.kernel,
    mesh=_MESH,
    out_type=jax.ShapeDtypeStruct((L, 8, 32, 8, 128), jnp.float32),
    scratch_types=[
        pltpu.VMEM((128,), jnp.int32),
        pltpu.VMEM((128,), jnp.int32),
        pltpu.VMEM((128,), jnp.int32),
        pltpu.VMEM((128,), jnp.int32),
        pltpu.VMEM((128, 128), jnp.float32),
        pltpu.VMEM((128, 128), jnp.float32),
        pltpu.VMEM((8, 8, 128), jnp.float32),
        pltpu.VMEM((8, 8, 128), jnp.float32),
        pltpu.SemaphoreType.DMA,
        pltpu.SemaphoreType.DMA,
        pltpu.SemaphoreType.DMA,
        pltpu.SemaphoreType.DMA,
    ],
    compiler_params=_PARAMS,
)
def _k2(xT_hbm, scratch_hbm, out_hbm,
        idx0, idx1, pidx0, pidx1, pairs0, pairs1, tb0, tb1,
        gs0, gs1, os0, os1):
    wid = lax.axis_index("s") * NC + lax.axis_index("c")
    iota16 = lax.iota(jnp.int32, 16)

    def prep(l, idx_v, pidx_v, pairs_v, gsem):
        boff = pl.multiple_of(wid * 128, 128)
        pltpu.sync_copy(xT_hbm.at[l, pl.ds(boff, 128)], idx_v)
        for g in range(8):
            pidx_v[pl.ds(g * 16, 16)] = (
                lax.shift_right_logical(idx_v[pl.ds(g * 16, 16)], 1))
        pltpu.async_copy(scratch_hbm.at[pidx_v], pairs_v, gsem)

    def wait_gather(pairs_v, gsem):
        pltpu.make_async_copy(
            scratch_hbm.at[pl.ds(0, 128)], pairs_v, gsem).wait()

    def compact(idx_v, pairs_v, tbuf):
        for g in range(8):
            r16 = iota16 + g * 16
            col0 = (idx_v[pl.ds(g * 16, 16)] & 1) * 64
            for mb in range(8):
                vals = [plsc.load_gather(pairs_v, [r16, col0 + mb * 8 + j])
                        for j in range(8)]
                for j in range(8):
                    m = mb * 8 + j
                    tbuf[m // 8, m % 8, pl.ds(g * 16, 16)] = vals[j]

    def start_out(l, tbuf, sem):
        pltpu.async_copy(tbuf, out_hbm.at[l, :, wid, :, :], sem)

    def wait_out(tbuf, sem):
        pltpu.make_async_copy(tbuf, out_hbm.at[0, :, wid, :, :], sem).wait()

    prep(0, idx0, pidx0, pairs0, gs0)

    def body(ll, carry):
        l0 = 2 * ll
        l1 = 2 * ll + 1
        prep(l1, idx1, pidx1, pairs1, gs1)
        wait_gather(pairs0, gs0)

        @pl.when(ll > 0)
        def _():
            wait_out(tb0, os0)
        compact(idx0, pairs0, tb0)
        start_out(l0, tb0, os0)

        @pl.when(ll < (L // 2) - 1)
        def _():
            prep(l0 + 2, idx0, pidx0, pairs0, gs0)
        wait_gather(pairs1, gs1)

        @pl.when(ll > 0)
        def _():
            wait_out(tb1, os1)
        compact(idx1, pairs1, tb1)
        start_out(l1, tb1, os1)
        return carry

    lax.fori_loop(0, L // 2, body, 0)
    wait_out(tb0, os0)
    wait_out(tb1, os1)


def kernel(x, table):
    tT = table.T
    tail = tT[:, V - 128:]
    scratch = _k1(tT, tail)
    xT = x.astype(jnp.int32).T
    out5 = _k2(xT, scratch)
    return out5.transpose(2, 4, 0, 1, 3).reshape(B, L, D)


# speedup vs baseline: 2.2969x; 1.6178x over previous
import functools

import jax
import jax.numpy as jnp
from jax import lax
from jax.experimental import pallas as pl
from jax.experimental.pallas import tpu as pltpu
from jax.experimental.pallas import tpu_sc as plsc

DIM = 64
NC = 2
NS = 16
NW = NC * NS

B_TOTAL = 4096 * 200
PER_W = B_TOTAL // NW
CH = 128
KB = 4
MB = CH * KB
NMB = PER_W // MB


@functools.partial(
    pl.kernel,
    mesh=plsc.VectorSubcoreMesh(core_axis_name="c", subcore_axis_name="s"),
    out_type=jax.ShapeDtypeStruct((B_TOTAL, DIM), jnp.float32),
    scratch_types=[
        pltpu.VMEM((PER_W,), jnp.int32),
        pltpu.VMEM((MB, DIM), jnp.float32),
        pltpu.SemaphoreType.DMA,
    ],
    compiler_params=pltpu.CompilerParams(use_tc_tiling_on_sc=False),
)
def _emb_gather(x_hbm, table_hbm, out_hbm, idx_v, rows_v, gsem):
    wid = lax.axis_index("s") * NC + lax.axis_index("c")
    base = wid * PER_W
    pltpu.sync_copy(x_hbm.at[pl.ds(base, PER_W)], idx_v)

    def block(g, carry):
        copies = []
        for b in range(KB):
            c = pltpu.async_copy(
                table_hbm.at[idx_v.at[pl.ds(g * MB + b * CH, CH)]],
                rows_v.at[pl.ds(b * CH, CH)],
                gsem,
            )
            copies.append(c)
        for c in copies:
            c.wait()
        pltpu.sync_copy(rows_v, out_hbm.at[pl.ds(base + g * MB, MB)])
        return carry

    lax.fori_loop(0, NMB, block, 0)


def kernel(x, table):
    xf = x.reshape(-1).astype(jnp.int32)
    out = _emb_gather(xf, table)
    return out.reshape(x.shape + (DIM,))
